# baseline (speedup 1.0000x reference)
import functools

import jax
import jax.numpy as jnp
from jax import lax
from jax.experimental import pallas as pl
from jax.experimental.pallas import tpu as pltpu
from jax.experimental.pallas import tpu_sc as plsc

T = 1000
B = 16384
L = 16
NC, NS = 1, 16
NW = NC * NS
BPW = B // NW


def _sc_body(t_hbm, ab_hbm, s_hbm, s2_hbm, b_hbm, out_hbm,
             t_v, ab_v, s_v, s2_v, b_v, out_v, sem):
    wid = lax.axis_index("s") * NC + lax.axis_index("c")
    base = wid * BPW

    in_cps = [
        pltpu.async_copy(t_hbm.at[pl.ds(base, BPW)], t_v, sem),
        pltpu.async_copy(ab_hbm, ab_v, sem),
        pltpu.async_copy(s_hbm, s_v, sem),
        pltpu.async_copy(s2_hbm, s2_v, sem),
        pltpu.async_copy(b_hbm, b_v, sem),
    ]
    for c in in_cps:
        c.wait()

    def step(i, carry):
        off = i * L
        idx = t_v[pl.ds(off, L)]
        ab = plsc.load_gather(ab_v, [idx])
        s = plsc.load_gather(s_v, [idx])
        s2 = plsc.load_gather(s2_v, [idx])
        b = plsc.load_gather(b_v, [idx])
        snr = (ab * ab) / jnp.maximum(s2, jnp.float32(1e-20))
        out_v[0, pl.ds(off, L)] = ab
        out_v[1, pl.ds(off, L)] = s
        out_v[2, pl.ds(off, L)] = s2
        out_v[3, pl.ds(off, L)] = b
        out_v[4, pl.ds(off, L)] = snr
        return carry

    lax.fori_loop(0, BPW // L, step, 0, unroll=8)

    pltpu.async_copy(out_v, out_hbm.at[:, pl.ds(base, BPW)], sem).wait()


@jax.jit
def _run(t, alpha_bar, sigma, sigma_sq, beta):
    mesh = plsc.VectorSubcoreMesh(core_axis_name="c", subcore_axis_name="s",
                                  num_cores=NC)
    k = functools.partial(
        pl.kernel,
        mesh=mesh,
        out_type=jax.ShapeDtypeStruct((5, B), jnp.float32),
        scratch_types=[
            pltpu.VMEM((BPW,), jnp.int32),
            pltpu.VMEM((T,), jnp.float32),
            pltpu.VMEM((T,), jnp.float32),
            pltpu.VMEM((T,), jnp.float32),
            pltpu.VMEM((T,), jnp.float32),
            pltpu.VMEM((5, BPW), jnp.float32),
            pltpu.SemaphoreType.DMA,
        ],
        compiler_params=pltpu.CompilerParams(needs_layout_passes=False),
    )(_sc_body)
    return k(t, alpha_bar, sigma, sigma_sq, beta)


def kernel(t, alpha_bar, sigma, sigma_sq, beta):
    return _run(t.astype(jnp.int32), alpha_bar, sigma, sigma_sq, beta)
